# Initial kernel scaffold; baseline (speedup 1.0000x reference)
import functools

import jax
import jax.numpy as jnp
from jax import lax
from jax.experimental import pallas as pl
from jax.experimental.pallas import tpu as pltpu
from jax.experimental.pallas import tpu_sc as plsc

D = 128
B = 4096 * 50
CH = 128
NROWS = B // CH
NW = 32
RPW = NROWS // NW

_mesh = plsc.VectorSubcoreMesh(core_axis_name="c", subcore_axis_name="s")


@functools.partial(
    pl.kernel,
    mesh=_mesh,
    out_type=jax.ShapeDtypeStruct((B, D), jnp.float32),
    scratch_types=[
        pltpu.VMEM((RPW, CH), jnp.int32),
        pltpu.VMEM((CH, D), jnp.float32),
        pltpu.SemaphoreType.DMA,
    ],
)
def _gather_kernel(idx_hbm, table_hbm, out_hbm, idx_v, rows_v, sem):
    wid = lax.axis_index("s") * 2 + lax.axis_index("c")
    row_base = wid * RPW
    pltpu.sync_copy(idx_hbm.at[pl.ds(row_base, RPW)], idx_v)

    def body(j, carry):
        pltpu.async_copy(table_hbm.at[idx_v.at[j]], rows_v, sem).wait()
        pltpu.sync_copy(rows_v, out_hbm.at[pl.ds((row_base + j) * CH, CH)])
        return carry

    lax.fori_loop(0, RPW, body, 0)


def kernel(input, weight):
    idx = input.reshape(NROWS, CH)
    out = _gather_kernel(idx, weight)
    return out.reshape(input.shape[0], input.shape[1], D)


# speedup vs baseline: 2.9680x; 2.9680x over previous
import functools

import jax
import jax.numpy as jnp
from jax import lax
from jax.experimental import pallas as pl
from jax.experimental.pallas import tpu as pltpu
from jax.experimental.pallas import tpu_sc as plsc

D = 128
B = 4096 * 50
CH = 128
NROWS = B // CH
NW = 32
RPW = NROWS // NW

_mesh = plsc.VectorSubcoreMesh(core_axis_name="c", subcore_axis_name="s")


@functools.partial(
    pl.kernel,
    mesh=_mesh,
    out_type=jax.ShapeDtypeStruct((B, D), jnp.float32),
    scratch_types=[
        pltpu.VMEM((RPW, CH), jnp.int32),
        pltpu.VMEM((CH, D), jnp.float32),
        pltpu.SemaphoreType.DMA,
    ],
)
def _gather_kernel(idx_hbm, table_hbm, out_hbm, idx_v, rows_v, sem):
    wid = lax.axis_index("s") * 2 + lax.axis_index("c")
    row_base = wid * RPW
    pltpu.sync_copy(idx_hbm.at[wid], idx_v)

    def body(j, carry):
        pltpu.async_copy(table_hbm.at[idx_v.at[j]], rows_v, sem).wait()
        pltpu.sync_copy(rows_v, out_hbm.at[pl.ds((row_base + j) * CH, CH)])
        return carry

    lax.fori_loop(0, RPW, body, 0)


def kernel(input, weight):
    idx = input.reshape(NW, RPW, CH)
    out = _gather_kernel(idx, weight)
    return out.reshape(input.shape[0], input.shape[1], D)


# speedup vs baseline: 3.3240x; 1.1199x over previous
import functools

import jax
import jax.numpy as jnp
from jax import lax
from jax.experimental import pallas as pl
from jax.experimental.pallas import tpu as pltpu
from jax.experimental.pallas import tpu_sc as plsc

D = 128
B = 4096 * 50
CH = 128
NROWS = B // CH
NW = 32
RPW = NROWS // NW

_mesh = plsc.VectorSubcoreMesh(core_axis_name="c", subcore_axis_name="s")


@functools.partial(
    pl.kernel,
    mesh=_mesh,
    out_type=jax.ShapeDtypeStruct((B, D), jnp.float32),
    scratch_types=[
        pltpu.VMEM((RPW, CH), jnp.int32),
        pltpu.VMEM((CH, D), jnp.float32),
        pltpu.VMEM((CH, D), jnp.float32),
        pltpu.SemaphoreType.DMA,
        pltpu.SemaphoreType.DMA,
    ],
)
def _gather_kernel(idx_hbm, table_hbm, out_hbm, idx_v, buf0, buf1, g0, g1):
    wid = lax.axis_index("s") * 2 + lax.axis_index("c")
    row_base = wid * RPW
    pltpu.sync_copy(idx_hbm.at[wid], idx_v)

    def gather(j, buf, sem):
        return pltpu.make_async_copy(table_hbm.at[idx_v.at[j]], buf, sem)

    def writeback(j, buf):
        pltpu.sync_copy(buf, out_hbm.at[pl.ds((row_base + j) * CH, CH)])

    gather(0, buf0, g0).start()

    def body(i, carry):
        j0 = 2 * i
        gather(j0 + 1, buf1, g1).start()
        gather(j0, buf0, g0).wait()
        writeback(j0, buf0)

        @pl.when(i < RPW // 2 - 1)
        def _():
            gather(j0 + 2, buf0, g0).start()

        gather(j0 + 1, buf1, g1).wait()
        writeback(j0 + 1, buf1)
        return carry

    lax.fori_loop(0, RPW // 2, body, 0)


def kernel(input, weight):
    idx = input.reshape(NW, RPW, CH)
    out = _gather_kernel(idx, weight)
    return out.reshape(input.shape[0], input.shape[1], D)
